# baseline (speedup 1.0000x reference)
import functools

import jax
import jax.numpy as jnp
from jax import lax
from jax.experimental import pallas as pl
from jax.experimental.pallas import tpu as pltpu
from jax.experimental.pallas import tpu_sc as plsc

VOCAB = 100000
EMB = 128
HID = 128
CTX = 200

NC = 2
NS = 16
NW = NC * NS
ROWS_PER_W = 8
CTX_PAD = NW * ROWS_PER_W

GBLK = 4000
NBLK = VOCAB // GBLK


def _sc_gather_sum(idx_pad, table):
    mesh = plsc.VectorSubcoreMesh(core_axis_name="c", subcore_axis_name="s")

    @functools.partial(
        pl.kernel,
        mesh=mesh,
        out_type=jax.ShapeDtypeStruct((NW, EMB), jnp.float32),
        scratch_types=[
            pltpu.VMEM((ROWS_PER_W,), jnp.int32),
            pltpu.VMEM((ROWS_PER_W, EMB), jnp.float32),
            pltpu.VMEM((EMB,), jnp.float32),
            pltpu.SemaphoreType.DMA,
        ],
    )
    def k(idx_hbm, table_hbm, out_hbm, idx_v, rows_v, acc_v, sem):
        wid = lax.axis_index("s") * NC + lax.axis_index("c")
        base = wid * ROWS_PER_W
        pltpu.sync_copy(idx_hbm.at[pl.ds(base, ROWS_PER_W)], idx_v)
        pltpu.async_copy(table_hbm.at[idx_v], rows_v, sem).wait()
        for c in range(EMB // 16):
            acc = jnp.zeros((16,), jnp.float32)
            for j in range(ROWS_PER_W):
                w = (base + j < CTX).astype(jnp.float32)
                acc = acc + rows_v[j, pl.ds(c * 16, 16)] * w
            acc_v[pl.ds(c * 16, 16)] = acc
        pltpu.sync_copy(acc_v, out_hbm.at[wid])

    return k(idx_pad, table)


def _logits_body(p_ref, w1_ref, b1_ref, w2_ref, b2_ref,
                 logits_ref, lse_ref, h_ref, m_ref, s_ref):
    i = pl.program_id(0)

    @pl.when(i == 0)
    def _():
        e = jnp.sum(p_ref[...], axis=0, keepdims=True)
        e8 = jnp.broadcast_to(e, (8, EMB))
        h = lax.dot_general(e8, w1_ref[...], (((1,), (1,)), ((), ())),
                            preferred_element_type=jnp.float32)
        h_ref[...] = jnp.maximum(h + b1_ref[...], 0.0)
        m_ref[0, 0] = -jnp.inf
        s_ref[0, 0] = 0.0

    lg = lax.dot_general(h_ref[...], w2_ref[0], (((1,), (1,)), ((), ())),
                         preferred_element_type=jnp.float32)
    lg = lg[0:1, :] + b2_ref[0]
    logits_ref[0] = lg

    m_old = m_ref[0, 0]
    m_new = jnp.maximum(m_old, jnp.max(lg))
    s_ref[0, 0] = s_ref[0, 0] * jnp.exp(m_old - m_new) + jnp.sum(jnp.exp(lg - m_new))
    m_ref[0, 0] = m_new

    @pl.when(i == NBLK - 1)
    def _():
        lse_ref[0, 0] = m_ref[0, 0] + jnp.log(s_ref[0, 0])


def _tc_logits(partials, W1, b1r, W2r, b2r):
    return pl.pallas_call(
        _logits_body,
        grid=(NBLK,),
        in_specs=[
            pl.BlockSpec((NW, EMB), lambda i: (0, 0)),
            pl.BlockSpec((HID, EMB), lambda i: (0, 0)),
            pl.BlockSpec((1, HID), lambda i: (0, 0)),
            pl.BlockSpec((1, GBLK, EMB), lambda i: (i, 0, 0)),
            pl.BlockSpec((1, 1, GBLK), lambda i: (i, 0, 0)),
        ],
        out_specs=[
            pl.BlockSpec((1, 1, GBLK), lambda i: (i, 0, 0)),
            pl.BlockSpec(memory_space=pltpu.SMEM, block_shape=(1, 1),
                         index_map=lambda i: (0, 0)),
        ],
        out_shape=[
            jax.ShapeDtypeStruct((NBLK, 1, GBLK), jnp.float32),
            jax.ShapeDtypeStruct((1, 1), jnp.float32),
        ],
        scratch_shapes=[
            pltpu.VMEM((8, HID), jnp.float32),
            pltpu.SMEM((1, 1), jnp.float32),
            pltpu.SMEM((1, 1), jnp.float32),
        ],
    )(partials, W1, b1r, W2r, b2r)


def _norm_body(lg_ref, lse_ref, out_ref):
    out_ref[0] = lg_ref[0] - lse_ref[0, 0]


def _tc_norm(logits, lse):
    return pl.pallas_call(
        _norm_body,
        grid=(NBLK,),
        in_specs=[
            pl.BlockSpec((1, 1, GBLK), lambda i: (i, 0, 0)),
            pl.BlockSpec(memory_space=pltpu.SMEM, block_shape=(1, 1),
                         index_map=lambda i: (0, 0)),
        ],
        out_specs=pl.BlockSpec((1, 1, GBLK), lambda i: (i, 0, 0)),
        out_shape=jax.ShapeDtypeStruct((NBLK, 1, GBLK), jnp.float32),
    )(logits, lse)


def kernel(inputs, emb_table, W1, b1, W2, b2):
    idx = inputs.astype(jnp.int32)
    idx_pad = jnp.concatenate([idx, jnp.zeros((CTX_PAD - CTX,), jnp.int32)])
    partials = _sc_gather_sum(idx_pad, emb_table)
    logits, lse = _tc_logits(
        partials,
        W1,
        b1.reshape(1, HID),
        W2.reshape(NBLK, GBLK, EMB),
        b2.reshape(NBLK, 1, GBLK),
    )
    out = _tc_norm(logits, lse)
    return out.reshape(1, VOCAB)
